# baseline (speedup 1.0000x reference)
import math

import jax
import jax.numpy as jnp
from jax.experimental import pallas as pl

IN_CH = 128
HIDDEN = [128, 128, 128]
POOL_RATIOS = [1.0, 0.5, 0.5]
EPS = 1e-5


def _bn(x, g, b):
    m = jnp.mean(x, axis=0)
    v = jnp.var(x, axis=0)
    return (x - m) / jnp.sqrt(v + EPS) * g + b


def _gcn(x, row, col, ew, W, b):
    N = x.shape[0]
    sl = jnp.arange(N, dtype=row.dtype)
    row2 = jnp.concatenate([row, sl])
    col2 = jnp.concatenate([col, sl])
    ew2 = jnp.concatenate([ew, jnp.ones((N,), x.dtype)])
    deg = jnp.zeros((N,), x.dtype).at[col2].add(ew2)
    dinv = jnp.where(deg > 0, 1.0 / jnp.sqrt(deg + 1e-12), 0.0)
    norm = dinv[row2] * ew2 * dinv[col2]
    h = x @ W
    out = jnp.zeros_like(h).at[col2].add(norm[:, None] * h[row2])
    return out + b


def kernel(x, edge_index, params):
    row = edge_index[0]
    col = edge_index[1]
    ew = jnp.ones((row.shape[0],), x.dtype)
    for i, ratio in enumerate(POOL_RATIOS):
        x = x @ params['enc%d_W1' % i] + params['enc%d_b1' % i]
        x = _bn(x, params['enc%d_g1' % i], params['enc%d_be1' % i])
        x = jax.nn.relu(x)
        x = _gcn(x, row, col, ew, params['enc%d_Wc' % i], params['enc%d_bc' % i])
        x = _bn(x, params['enc%d_g2' % i], params['enc%d_be2' % i])
        x = jax.nn.relu(x)
        N = x.shape[0]
        p = params['enc%d_p' % i]
        score = x @ p / jnp.linalg.norm(p)
        k = int(math.ceil(ratio * N))
        vals, perm = jax.lax.top_k(score, k)
        x = x[perm] * jnp.tanh(vals)[:, None]
        keep = jnp.zeros((N,), bool).at[perm].set(True)
        newidx = jnp.zeros((N,), row.dtype).at[perm].set(jnp.arange(k, dtype=row.dtype))
        valid = keep[row] & keep[col]
        row = jnp.where(valid, newidx[row], 0)
        col = jnp.where(valid, newidx[col], 0)
        ew = jnp.where(valid, ew, jnp.zeros_like(ew))
    x = jnp.mean(x, axis=0, keepdims=True)
    x = jax.nn.relu(x @ params['dec2_W'] + params['dec2_b'])
    x = jax.nn.relu(x @ params['dec1_W'] + params['dec1_b'])
    x = x @ params['dec0_W'] + params['dec0_b']
    return x.reshape(1, -1, 5)


# speedup vs baseline: 4.2674x; 4.2674x over previous
import functools

import jax
import jax.numpy as jnp
from jax import lax
from jax.experimental import pallas as pl
from jax.experimental.pallas import tpu as pltpu
from jax.experimental.pallas import tpu_sc as plsc

N0 = 10000
NP = 10240
E = 320000
EPS = 1e-5
DUMMY = N0

NC, NS, L = 2, 16, 16
NW = NC * NS
EP = E // NW
BLK = 512
GRID = NP // BLK
SL = NP // NS

_mesh = plsc.VectorSubcoreMesh(core_axis_name="c", subcore_axis_name="s")
_sc_params = pltpu.CompilerParams(needs_layout_passes=False)


_C2 = 2000


@functools.partial(
    pl.kernel, mesh=_mesh,
    out_type=[jax.ShapeDtypeStruct((E,), jnp.int32),
              jax.ShapeDtypeStruct((NC, NP), jnp.float32)],
    scratch_types=[pltpu.VMEM((NP,), jnp.float32),
                   pltpu.VMEM((_C2,), jnp.int32),
                   pltpu.VMEM((_C2,), jnp.int32),
                   pltpu.VMEM((_C2,), jnp.int32),
                   pltpu.VMEM((_C2,), jnp.float32),
                   pltpu.VMEM((SL,), jnp.float32),
                   pltpu.VMEM_SHARED((NP,), jnp.float32)],
    compiler_params=_sc_params,
)
def sc_edge(row_hbm, col_hbm, alive_hbm, rowg_hbm, degp_hbm,
            alive_t, row_v, col_v, rowg_v, ew_v, zero_v, deg_sh):
    c = lax.axis_index("c")
    s = lax.axis_index("s")
    wid = s * NC + c
    base = wid * EP

    pltpu.sync_copy(alive_hbm, alive_t)
    for j in range(SL // L):
        zero_v[pl.ds(j * L, L)] = jnp.zeros((L,), jnp.float32)
    pltpu.sync_copy(zero_v, deg_sh.at[pl.ds(s * SL, SL)])
    plsc.subcore_barrier()

    def chunk(k, _):
        off = base + k * _C2
        pltpu.sync_copy(row_hbm.at[pl.ds(off, _C2)], row_v)
        pltpu.sync_copy(col_hbm.at[pl.ds(off, _C2)], col_v)

        def vec(i, _):
            rv = row_v[pl.ds(i * L, L)]
            cv = col_v[pl.ds(i * L, L)]
            ar = plsc.load_gather(alive_t, [rv])
            ac = plsc.load_gather(alive_t, [cv])
            ok = jnp.logical_and(ar > 0.0, ac > 0.0)
            rowg_v[pl.ds(i * L, L)] = jnp.where(ok, rv, DUMMY)
            ew_v[pl.ds(i * L, L)] = jnp.where(ok, 1.0, 0.0)
            return 0

        lax.fori_loop(0, _C2 // L, vec, 0)
        pltpu.sync_copy(rowg_v, rowg_hbm.at[pl.ds(off, _C2)])
        pltpu.sync_copy(ew_v, deg_sh.at[col_v], add=True)
        return 0

    lax.fori_loop(0, EP // _C2, chunk, 0)
    plsc.subcore_barrier()
    pltpu.sync_copy(deg_sh.at[pl.ds(s * SL, SL)],
                    degp_hbm.at[c, pl.ds(s * SL, SL)])


_CM = 200


@functools.partial(
    pl.kernel, mesh=_mesh,
    out_type=jax.ShapeDtypeStruct((NC, NP, 128), jnp.float32),
    scratch_types=[pltpu.VMEM((_CM,), jnp.int32),
                   pltpu.VMEM((_CM,), jnp.int32),
                   pltpu.VMEM((_CM, 128), jnp.float32),
                   pltpu.VMEM((32, 128), jnp.float32),
                   pltpu.VMEM_SHARED((NP, 128), jnp.float32),
                   pltpu.SemaphoreType.DMA],
    compiler_params=_sc_params,
)
def sc_msg(rowg_hbm, col_hbm, g2_hbm, outp_hbm,
           rowg_v, col_v, rows_v, zero_v, acc_sh, sem):
    c = lax.axis_index("c")
    s = lax.axis_index("s")
    wid = s * NC + c
    base = wid * EP

    for i in range(32):
        for j in range(8):
            zero_v[i, pl.ds(j * L, L)] = jnp.zeros((L,), jnp.float32)

    def zrow(jj, _):
        pltpu.sync_copy(zero_v, acc_sh.at[pl.ds(s * SL + jj * 32, 32), :])
        return 0

    lax.fori_loop(0, SL // 32, zrow, 0)
    plsc.subcore_barrier()

    def chunk(k, _):
        off = base + k * _CM
        pltpu.sync_copy(rowg_hbm.at[pl.ds(off, _CM)], rowg_v)
        pltpu.sync_copy(col_hbm.at[pl.ds(off, _CM)], col_v)
        pltpu.async_copy(g2_hbm.at[rowg_v], rows_v, sem).wait()
        pltpu.sync_copy(rows_v, acc_sh.at[col_v], add=True)
        return 0

    lax.fori_loop(0, EP // _CM, chunk, 0)
    plsc.subcore_barrier()
    pltpu.sync_copy(acc_sh.at[pl.ds(s * SL, SL), :],
                    outp_hbm.at[c, pl.ds(s * SL, SL), :])


def _rows_iota():
    return lax.broadcasted_iota(jnp.int32, (BLK, 1), 0)


def _make_k1(first, n_alive):
    def body(x_ref, al_ref, sc_ref, thr_ref, w_ref, b_ref,
             y_ref, st_ref, alout_ref):
        i = pl.program_id(0)
        rows = _rows_iota() + i * BLK
        if first:
            alive = (rows < N0).astype(jnp.float32)
            xs = x_ref[...]
        else:
            sel = jnp.logical_and(al_ref[...] > 0.0,
                                  sc_ref[...] >= thr_ref[0, 0])
            alive = sel.astype(jnp.float32)
            xs = jnp.where(sel, jnp.tanh(sc_ref[...]), 0.0) * x_ref[...]
        y = jnp.dot(xs, w_ref[...], preferred_element_type=jnp.float32)
        y = y + b_ref[...]
        y_ref[...] = y
        alout_ref[...] = alive
        ym = alive * y
        part = jnp.concatenate([jnp.sum(ym, axis=0, keepdims=True),
                                jnp.sum(ym * ym, axis=0, keepdims=True)],
                               axis=0)

        @pl.when(i == 0)
        def _():
            st_ref[...] = part

        @pl.when(i > 0)
        def _():
            st_ref[...] += part

    return pl.pallas_call(
        body,
        grid=(GRID,),
        in_specs=[
            pl.BlockSpec((BLK, 128), lambda i: (i, 0)),
            pl.BlockSpec((BLK, 1), lambda i: (i, 0)),
            pl.BlockSpec((BLK, 1), lambda i: (i, 0)),
            pl.BlockSpec((1, 1), lambda i: (0, 0)),
            pl.BlockSpec((128, 128), lambda i: (0, 0)),
            pl.BlockSpec((1, 128), lambda i: (0, 0)),
        ],
        out_specs=[
            pl.BlockSpec((BLK, 128), lambda i: (i, 0)),
            pl.BlockSpec((2, 128), lambda i: (0, 0)),
            pl.BlockSpec((BLK, 1), lambda i: (i, 0)),
        ],
        out_shape=[
            jax.ShapeDtypeStruct((NP, 128), jnp.float32),
            jax.ShapeDtypeStruct((2, 128), jnp.float32),
            jax.ShapeDtypeStruct((NP, 1), jnp.float32),
        ],
        compiler_params=pltpu.CompilerParams(
            dimension_semantics=("arbitrary",)),
    )


def _dinv_block(degp):
    deg = degp[0] + degp[1] + 1.0
    return 1.0 / jnp.sqrt(deg + 1e-12)


def _make_k2(n_alive):
    def body(y_ref, st_ref, al_ref, dp_ref, g_ref, be_ref, wc_ref, g2_ref):
        m = st_ref[0:1, :] / n_alive
        var = st_ref[1:2, :] / n_alive - m * m
        rstd = 1.0 / jnp.sqrt(var + EPS)
        x1 = jnp.maximum((y_ref[...] - m) * rstd * g_ref[...] + be_ref[...],
                         0.0)
        h = jnp.dot(x1, wc_ref[...], preferred_element_type=jnp.float32)
        g2 = _dinv_block(dp_ref[...]) * h
        g2_ref[...] = jnp.where(al_ref[...] > 0.0, g2, 0.0)

    return pl.pallas_call(
        body,
        grid=(GRID,),
        in_specs=[
            pl.BlockSpec((BLK, 128), lambda i: (i, 0)),
            pl.BlockSpec((2, 128), lambda i: (0, 0)),
            pl.BlockSpec((BLK, 1), lambda i: (i, 0)),
            pl.BlockSpec((2, BLK, 1), lambda i: (0, i, 0)),
            pl.BlockSpec((1, 128), lambda i: (0, 0)),
            pl.BlockSpec((1, 128), lambda i: (0, 0)),
            pl.BlockSpec((128, 128), lambda i: (0, 0)),
        ],
        out_specs=pl.BlockSpec((BLK, 128), lambda i: (i, 0)),
        out_shape=jax.ShapeDtypeStruct((NP, 128), jnp.float32),
        compiler_params=pltpu.CompilerParams(
            dimension_semantics=("arbitrary",)),
    )


def _make_k3():
    def body(op_ref, g2_ref, al_ref, dp_ref, bc_ref, z_ref, st_ref):
        i = pl.program_id(0)
        agg = op_ref[0] + op_ref[1] + g2_ref[...]
        z = _dinv_block(dp_ref[...]) * agg + bc_ref[...]
        z = jnp.where(al_ref[...] > 0.0, z, 0.0)
        z_ref[...] = z
        part = jnp.concatenate([jnp.sum(z, axis=0, keepdims=True),
                                jnp.sum(z * z, axis=0, keepdims=True)],
                               axis=0)

        @pl.when(i == 0)
        def _():
            st_ref[...] = part

        @pl.when(i > 0)
        def _():
            st_ref[...] += part

    return pl.pallas_call(
        body,
        grid=(GRID,),
        in_specs=[
            pl.BlockSpec((2, BLK, 128), lambda i: (0, i, 0)),
            pl.BlockSpec((BLK, 128), lambda i: (i, 0)),
            pl.BlockSpec((BLK, 1), lambda i: (i, 0)),
            pl.BlockSpec((2, BLK, 1), lambda i: (0, i, 0)),
            pl.BlockSpec((1, 128), lambda i: (0, 0)),
        ],
        out_specs=[
            pl.BlockSpec((BLK, 128), lambda i: (i, 0)),
            pl.BlockSpec((2, 128), lambda i: (0, 0)),
        ],
        out_shape=[
            jax.ShapeDtypeStruct((NP, 128), jnp.float32),
            jax.ShapeDtypeStruct((2, 128), jnp.float32),
        ],
        compiler_params=pltpu.CompilerParams(
            dimension_semantics=("arbitrary",)),
    )


def _make_k4(n_alive):
    def body(z_ref, st_ref, al_ref, g_ref, be_ref, p_ref, x2_ref, sc_ref):
        m = st_ref[0:1, :] / n_alive
        var = st_ref[1:2, :] / n_alive - m * m
        rstd = 1.0 / jnp.sqrt(var + EPS)
        x2 = jnp.maximum((z_ref[...] - m) * rstd * g_ref[...] + be_ref[...],
                         0.0)
        alive = al_ref[...] > 0.0
        x2 = jnp.where(alive, x2, 0.0)
        x2_ref[...] = x2
        pn = p_ref[...] / jnp.sqrt(jnp.sum(p_ref[...] * p_ref[...]))
        s = jnp.sum(x2 * pn, axis=1, keepdims=True)
        sc_ref[...] = jnp.where(alive, s, -1e30)

    return pl.pallas_call(
        body,
        grid=(GRID,),
        in_specs=[
            pl.BlockSpec((BLK, 128), lambda i: (i, 0)),
            pl.BlockSpec((2, 128), lambda i: (0, 0)),
            pl.BlockSpec((BLK, 1), lambda i: (i, 0)),
            pl.BlockSpec((1, 128), lambda i: (0, 0)),
            pl.BlockSpec((1, 128), lambda i: (0, 0)),
            pl.BlockSpec((1, 128), lambda i: (0, 0)),
        ],
        out_specs=[
            pl.BlockSpec((BLK, 128), lambda i: (i, 0)),
            pl.BlockSpec((BLK, 1), lambda i: (i, 0)),
        ],
        out_shape=[
            jax.ShapeDtypeStruct((NP, 128), jnp.float32),
            jax.ShapeDtypeStruct((NP, 1), jnp.float32),
        ],
        compiler_params=pltpu.CompilerParams(
            dimension_semantics=("arbitrary",)),
    )


def _make_k5(n_final):
    def body(x2_ref, sc_ref, thr_ref, w2_ref, b2_ref, w1_ref, b1_ref,
             w0_ref, b0_ref, out_ref):
        sel = sc_ref[...] >= thr_ref[0, 0]
        w = jnp.where(sel, jnp.tanh(sc_ref[...]), 0.0)
        xm = jnp.sum(w * x2_ref[...], axis=0, keepdims=True) / n_final
        h = jnp.maximum(jnp.dot(xm, w2_ref[...],
                                preferred_element_type=jnp.float32)
                        + b2_ref[...], 0.0)
        h = jnp.maximum(jnp.dot(h, w1_ref[...],
                                preferred_element_type=jnp.float32)
                        + b1_ref[...], 0.0)
        out_ref[...] = jnp.dot(h, w0_ref[...],
                               preferred_element_type=jnp.float32) + b0_ref[...]

    return pl.pallas_call(
        body,
        out_shape=jax.ShapeDtypeStruct((1, 3750), jnp.float32),
        compiler_params=pltpu.CompilerParams(
            vmem_limit_bytes=100 * 1024 * 1024),
    )


_K1F = _make_k1(True, 10000.0)
_K1A = _make_k1(False, 10000.0)
_K1B = _make_k1(False, 5000.0)
_K2 = {10000.0: _make_k2(10000.0), 5000.0: _make_k2(5000.0)}
_K3 = _make_k3()
_K4 = {10000.0: _make_k4(10000.0), 5000.0: _make_k4(5000.0)}
_K5 = _make_k5(2500.0)

_ZPAD = (0, NP - N0)


def kernel(x, edge_index, params):
    row = edge_index[0].astype(jnp.int32)
    col = edge_index[1].astype(jnp.int32)
    xp = jnp.pad(x, (_ZPAD, (0, 0)))
    alive0 = jnp.pad(jnp.ones((N0, 1), jnp.float32), (_ZPAD, (0, 0)))

    rowg01, degp01 = sc_edge(row, col, alive0.reshape(NP))

    zero_s = jnp.zeros((NP, 1), jnp.float32)
    neg_thr = jnp.full((1, 1), -1e29, jnp.float32)

    xcur = xp
    alive = alive0
    score = zero_s
    thr = neg_thr
    rowg, degp = rowg01, degp01
    n_alive = [10000.0, 10000.0, 5000.0]

    for i in range(3):
        na = n_alive[i]
        k1 = _K1F if i == 0 else (_K1A if i == 1 else _K1B)
        y, st1, alive = k1(xcur, alive, score, thr,
                           params['enc%d_W1' % i],
                           params['enc%d_b1' % i].reshape(1, 128))
        if i == 2:
            rowg, degp = sc_edge(row, col, alive.reshape(NP))
        degp3 = degp.reshape(2, NP, 1)
        g2 = _K2[na](y, st1, alive, degp3,
                     params['enc%d_g1' % i].reshape(1, 128),
                     params['enc%d_be1' % i].reshape(1, 128),
                     params['enc%d_Wc' % i])
        outp = sc_msg(rowg, col, g2)
        z, st2 = _K3(outp, g2, alive, degp3,
                     params['enc%d_bc' % i].reshape(1, 128))
        x2, score = _K4[na](z, st2, alive,
                            params['enc%d_g2' % i].reshape(1, 128),
                            params['enc%d_be2' % i].reshape(1, 128),
                            params['enc%d_p' % i].reshape(1, 128))
        if i < 2:
            kk = 5000 if i == 1 else 2500
            if i == 0:
                thr = neg_thr
            else:
                vals, _ = lax.top_k(score.reshape(NP), kk)
                thr = vals[kk - 1].reshape(1, 1)
        else:
            vals, _ = lax.top_k(score.reshape(NP), 2500)
            thr = vals[2499].reshape(1, 1)
        xcur = x2

    out = _K5(x2, score, thr,
              params['dec2_W'], params['dec2_b'].reshape(1, 256),
              params['dec1_W'], params['dec1_b'].reshape(1, 512),
              params['dec0_W'], params['dec0_b'].reshape(1, 3750))
    return out.reshape(1, 750, 5)


# speedup vs baseline: 44.1308x; 10.3413x over previous
import functools

import jax
import jax.numpy as jnp
from jax import lax
from jax.experimental import pallas as pl
from jax.experimental.pallas import tpu as pltpu
from jax.experimental.pallas import tpu_sc as plsc

N0 = 10000
NP = 10240
E = 320000
EPS = 1e-5
DUMMY = N0

NC, NS, L = 2, 16, 16
NW = NC * NS
EP = E // NW
BLK = 512
GRID = NP // BLK
SL = NP // NS

_mesh = plsc.VectorSubcoreMesh(core_axis_name="c", subcore_axis_name="s")
_sc_params = pltpu.CompilerParams(needs_layout_passes=False)


_C2 = 2000


@functools.partial(
    pl.kernel, mesh=_mesh,
    out_type=[jax.ShapeDtypeStruct((E,), jnp.int32),
              jax.ShapeDtypeStruct((NC, NP), jnp.float32)],
    scratch_types=[pltpu.VMEM((NP,), jnp.float32),
                   pltpu.VMEM((_C2,), jnp.int32),
                   pltpu.VMEM((_C2,), jnp.int32),
                   pltpu.VMEM((_C2,), jnp.int32),
                   pltpu.VMEM((_C2,), jnp.float32),
                   pltpu.VMEM((SL,), jnp.float32),
                   pltpu.VMEM_SHARED((NP,), jnp.float32)],
    compiler_params=_sc_params,
)
def sc_edge(row_hbm, col_hbm, alive_hbm, rowg_hbm, degp_hbm,
            alive_t, row_v, col_v, rowg_v, ew_v, zero_v, deg_sh):
    c = lax.axis_index("c")
    s = lax.axis_index("s")
    wid = s * NC + c
    base = wid * EP

    pltpu.sync_copy(alive_hbm, alive_t)
    for j in range(SL // L):
        zero_v[pl.ds(j * L, L)] = jnp.zeros((L,), jnp.float32)
    pltpu.sync_copy(zero_v, deg_sh.at[pl.ds(s * SL, SL)])
    plsc.subcore_barrier()

    def chunk(k, _):
        off = base + k * _C2
        pltpu.sync_copy(row_hbm.at[pl.ds(off, _C2)], row_v)
        pltpu.sync_copy(col_hbm.at[pl.ds(off, _C2)], col_v)

        def vec(i, _):
            rv = row_v[pl.ds(i * L, L)]
            cv = col_v[pl.ds(i * L, L)]
            ar = plsc.load_gather(alive_t, [rv])
            ac = plsc.load_gather(alive_t, [cv])
            ok = jnp.logical_and(ar > 0.0, ac > 0.0)
            iota = lax.broadcasted_iota(jnp.int32, (L,), 0)
            dummy = DUMMY + jnp.bitwise_and(iota + i * L, 127)
            rowg_v[pl.ds(i * L, L)] = jnp.where(ok, rv, dummy)
            ew_v[pl.ds(i * L, L)] = jnp.where(ok, 1.0, 0.0)
            return 0

        lax.fori_loop(0, _C2 // L, vec, 0)
        pltpu.sync_copy(rowg_v, rowg_hbm.at[pl.ds(off, _C2)])
        pltpu.sync_copy(ew_v, deg_sh.at[col_v], add=True)
        return 0

    lax.fori_loop(0, EP // _C2, chunk, 0)
    plsc.subcore_barrier()
    pltpu.sync_copy(deg_sh.at[pl.ds(s * SL, SL)],
                    degp_hbm.at[c, pl.ds(s * SL, SL)])


_CM = 200


@functools.partial(
    pl.kernel, mesh=_mesh,
    out_type=jax.ShapeDtypeStruct((NC, NP, 128), jnp.float32),
    scratch_types=[pltpu.VMEM((_CM,), jnp.int32),
                   pltpu.VMEM((_CM,), jnp.int32),
                   pltpu.VMEM((_CM, 128), jnp.float32),
                   pltpu.VMEM((32, 128), jnp.float32),
                   pltpu.VMEM_SHARED((NP, 128), jnp.float32),
                   pltpu.SemaphoreType.DMA],
    compiler_params=_sc_params,
)
def sc_msg(rowg_hbm, col_hbm, g2_hbm, outp_hbm,
           rowg_v, col_v, rows_v, zero_v, acc_sh, sem):
    c = lax.axis_index("c")
    s = lax.axis_index("s")
    wid = s * NC + c
    base = wid * EP

    for i in range(32):
        for j in range(8):
            zero_v[i, pl.ds(j * L, L)] = jnp.zeros((L,), jnp.float32)

    def zrow(jj, _):
        pltpu.sync_copy(zero_v, acc_sh.at[pl.ds(s * SL + jj * 32, 32), :])
        return 0

    lax.fori_loop(0, SL // 32, zrow, 0)
    plsc.subcore_barrier()

    def chunk(k, _):
        off = base + k * _CM
        pltpu.sync_copy(rowg_hbm.at[pl.ds(off, _CM)], rowg_v)
        pltpu.sync_copy(col_hbm.at[pl.ds(off, _CM)], col_v)
        pltpu.async_copy(g2_hbm.at[rowg_v], rows_v, sem).wait()
        pltpu.sync_copy(rows_v, acc_sh.at[col_v], add=True)
        return 0

    lax.fori_loop(0, EP // _CM, chunk, 0)
    plsc.subcore_barrier()
    pltpu.sync_copy(acc_sh.at[pl.ds(s * SL, SL), :],
                    outp_hbm.at[c, pl.ds(s * SL, SL), :])


def _rows_iota():
    return lax.broadcasted_iota(jnp.int32, (BLK, 1), 0)


def _make_k1(first, n_alive):
    def body(x_ref, al_ref, sc_ref, thr_ref, w_ref, b_ref,
             y_ref, st_ref, alout_ref):
        i = pl.program_id(0)
        rows = _rows_iota() + i * BLK
        if first:
            alive = (rows < N0).astype(jnp.float32)
            xs = x_ref[...]
        else:
            sel = jnp.logical_and(al_ref[...] > 0.0,
                                  sc_ref[...] >= thr_ref[0, 0])
            alive = sel.astype(jnp.float32)
            xs = jnp.where(sel, jnp.tanh(sc_ref[...]), 0.0) * x_ref[...]
        y = jnp.dot(xs, w_ref[...], preferred_element_type=jnp.float32)
        y = y + b_ref[...]
        y_ref[...] = y
        alout_ref[...] = alive
        ym = alive * y
        part = jnp.concatenate([jnp.sum(ym, axis=0, keepdims=True),
                                jnp.sum(ym * ym, axis=0, keepdims=True)],
                               axis=0)

        @pl.when(i == 0)
        def _():
            st_ref[...] = part

        @pl.when(i > 0)
        def _():
            st_ref[...] += part

    return pl.pallas_call(
        body,
        grid=(GRID,),
        in_specs=[
            pl.BlockSpec((BLK, 128), lambda i: (i, 0)),
            pl.BlockSpec((BLK, 1), lambda i: (i, 0)),
            pl.BlockSpec((BLK, 1), lambda i: (i, 0)),
            pl.BlockSpec((1, 1), lambda i: (0, 0)),
            pl.BlockSpec((128, 128), lambda i: (0, 0)),
            pl.BlockSpec((1, 128), lambda i: (0, 0)),
        ],
        out_specs=[
            pl.BlockSpec((BLK, 128), lambda i: (i, 0)),
            pl.BlockSpec((2, 128), lambda i: (0, 0)),
            pl.BlockSpec((BLK, 1), lambda i: (i, 0)),
        ],
        out_shape=[
            jax.ShapeDtypeStruct((NP, 128), jnp.float32),
            jax.ShapeDtypeStruct((2, 128), jnp.float32),
            jax.ShapeDtypeStruct((NP, 1), jnp.float32),
        ],
        compiler_params=pltpu.CompilerParams(
            dimension_semantics=("arbitrary",)),
    )


def _dinv_block(degp):
    deg = degp[0] + degp[1] + 1.0
    return 1.0 / jnp.sqrt(deg + 1e-12)


def _make_k2(n_alive):
    def body(y_ref, st_ref, al_ref, dp_ref, g_ref, be_ref, wc_ref, g2_ref):
        m = st_ref[0:1, :] / n_alive
        var = st_ref[1:2, :] / n_alive - m * m
        rstd = 1.0 / jnp.sqrt(var + EPS)
        x1 = jnp.maximum((y_ref[...] - m) * rstd * g_ref[...] + be_ref[...],
                         0.0)
        h = jnp.dot(x1, wc_ref[...], preferred_element_type=jnp.float32)
        g2 = _dinv_block(dp_ref[...]) * h
        g2_ref[...] = jnp.where(al_ref[...] > 0.0, g2, 0.0)

    return pl.pallas_call(
        body,
        grid=(GRID,),
        in_specs=[
            pl.BlockSpec((BLK, 128), lambda i: (i, 0)),
            pl.BlockSpec((2, 128), lambda i: (0, 0)),
            pl.BlockSpec((BLK, 1), lambda i: (i, 0)),
            pl.BlockSpec((2, BLK, 1), lambda i: (0, i, 0)),
            pl.BlockSpec((1, 128), lambda i: (0, 0)),
            pl.BlockSpec((1, 128), lambda i: (0, 0)),
            pl.BlockSpec((128, 128), lambda i: (0, 0)),
        ],
        out_specs=pl.BlockSpec((BLK, 128), lambda i: (i, 0)),
        out_shape=jax.ShapeDtypeStruct((NP, 128), jnp.float32),
        compiler_params=pltpu.CompilerParams(
            dimension_semantics=("arbitrary",)),
    )


def _make_k3():
    def body(op_ref, g2_ref, al_ref, dp_ref, bc_ref, z_ref, st_ref):
        i = pl.program_id(0)
        agg = op_ref[0] + op_ref[1] + g2_ref[...]
        z = _dinv_block(dp_ref[...]) * agg + bc_ref[...]
        z = jnp.where(al_ref[...] > 0.0, z, 0.0)
        z_ref[...] = z
        part = jnp.concatenate([jnp.sum(z, axis=0, keepdims=True),
                                jnp.sum(z * z, axis=0, keepdims=True)],
                               axis=0)

        @pl.when(i == 0)
        def _():
            st_ref[...] = part

        @pl.when(i > 0)
        def _():
            st_ref[...] += part

    return pl.pallas_call(
        body,
        grid=(GRID,),
        in_specs=[
            pl.BlockSpec((2, BLK, 128), lambda i: (0, i, 0)),
            pl.BlockSpec((BLK, 128), lambda i: (i, 0)),
            pl.BlockSpec((BLK, 1), lambda i: (i, 0)),
            pl.BlockSpec((2, BLK, 1), lambda i: (0, i, 0)),
            pl.BlockSpec((1, 128), lambda i: (0, 0)),
        ],
        out_specs=[
            pl.BlockSpec((BLK, 128), lambda i: (i, 0)),
            pl.BlockSpec((2, 128), lambda i: (0, 0)),
        ],
        out_shape=[
            jax.ShapeDtypeStruct((NP, 128), jnp.float32),
            jax.ShapeDtypeStruct((2, 128), jnp.float32),
        ],
        compiler_params=pltpu.CompilerParams(
            dimension_semantics=("arbitrary",)),
    )


def _make_k4(n_alive):
    def body(z_ref, st_ref, al_ref, g_ref, be_ref, p_ref, x2_ref, sc_ref):
        m = st_ref[0:1, :] / n_alive
        var = st_ref[1:2, :] / n_alive - m * m
        rstd = 1.0 / jnp.sqrt(var + EPS)
        x2 = jnp.maximum((z_ref[...] - m) * rstd * g_ref[...] + be_ref[...],
                         0.0)
        alive = al_ref[...] > 0.0
        x2 = jnp.where(alive, x2, 0.0)
        x2_ref[...] = x2
        pn = p_ref[...] / jnp.sqrt(jnp.sum(p_ref[...] * p_ref[...]))
        s = jnp.sum(x2 * pn, axis=1, keepdims=True)
        sc_ref[...] = jnp.where(alive, s, -1e30)

    return pl.pallas_call(
        body,
        grid=(GRID,),
        in_specs=[
            pl.BlockSpec((BLK, 128), lambda i: (i, 0)),
            pl.BlockSpec((2, 128), lambda i: (0, 0)),
            pl.BlockSpec((BLK, 1), lambda i: (i, 0)),
            pl.BlockSpec((1, 128), lambda i: (0, 0)),
            pl.BlockSpec((1, 128), lambda i: (0, 0)),
            pl.BlockSpec((1, 128), lambda i: (0, 0)),
        ],
        out_specs=[
            pl.BlockSpec((BLK, 128), lambda i: (i, 0)),
            pl.BlockSpec((BLK, 1), lambda i: (i, 0)),
        ],
        out_shape=[
            jax.ShapeDtypeStruct((NP, 128), jnp.float32),
            jax.ShapeDtypeStruct((NP, 1), jnp.float32),
        ],
        compiler_params=pltpu.CompilerParams(
            dimension_semantics=("arbitrary",)),
    )


def _make_k5(n_final):
    def body(x2_ref, sc_ref, thr_ref, w2_ref, b2_ref, w1_ref, b1_ref,
             w0_ref, b0_ref, out_ref):
        sel = sc_ref[...] >= thr_ref[0, 0]
        w = jnp.where(sel, jnp.tanh(sc_ref[...]), 0.0)
        xm = jnp.sum(w * x2_ref[...], axis=0, keepdims=True) / n_final
        h = jnp.maximum(jnp.dot(xm, w2_ref[...],
                                preferred_element_type=jnp.float32)
                        + b2_ref[...], 0.0)
        h = jnp.maximum(jnp.dot(h, w1_ref[...],
                                preferred_element_type=jnp.float32)
                        + b1_ref[...], 0.0)
        out_ref[...] = jnp.dot(h, w0_ref[...],
                               preferred_element_type=jnp.float32) + b0_ref[...]

    return pl.pallas_call(
        body,
        out_shape=jax.ShapeDtypeStruct((1, 3750), jnp.float32),
        compiler_params=pltpu.CompilerParams(
            vmem_limit_bytes=100 * 1024 * 1024),
    )


_K1F = _make_k1(True, 10000.0)
_K1A = _make_k1(False, 10000.0)
_K1B = _make_k1(False, 5000.0)
_K2 = {10000.0: _make_k2(10000.0), 5000.0: _make_k2(5000.0)}
_K3 = _make_k3()
_K4 = {10000.0: _make_k4(10000.0), 5000.0: _make_k4(5000.0)}
_K5 = _make_k5(2500.0)

_ZPAD = (0, NP - N0)


def kernel(x, edge_index, params):
    row = edge_index[0].astype(jnp.int32)
    col = edge_index[1].astype(jnp.int32)
    xp = jnp.pad(x, (_ZPAD, (0, 0)))
    alive0 = jnp.pad(jnp.ones((N0, 1), jnp.float32), (_ZPAD, (0, 0)))

    rowg01, degp01 = sc_edge(row, col, alive0.reshape(NP))

    zero_s = jnp.zeros((NP, 1), jnp.float32)
    neg_thr = jnp.full((1, 1), -1e29, jnp.float32)

    xcur = xp
    alive = alive0
    score = zero_s
    thr = neg_thr
    rowg, degp = rowg01, degp01
    n_alive = [10000.0, 10000.0, 5000.0]

    for i in range(3):
        na = n_alive[i]
        k1 = _K1F if i == 0 else (_K1A if i == 1 else _K1B)
        y, st1, alive = k1(xcur, alive, score, thr,
                           params['enc%d_W1' % i],
                           params['enc%d_b1' % i].reshape(1, 128))
        if i == 2:
            rowg, degp = sc_edge(row, col, alive.reshape(NP))
        degp3 = degp.reshape(2, NP, 1)
        g2 = _K2[na](y, st1, alive, degp3,
                     params['enc%d_g1' % i].reshape(1, 128),
                     params['enc%d_be1' % i].reshape(1, 128),
                     params['enc%d_Wc' % i])
        outp = sc_msg(rowg, col, g2)
        z, st2 = _K3(outp, g2, alive, degp3,
                     params['enc%d_bc' % i].reshape(1, 128))
        x2, score = _K4[na](z, st2, alive,
                            params['enc%d_g2' % i].reshape(1, 128),
                            params['enc%d_be2' % i].reshape(1, 128),
                            params['enc%d_p' % i].reshape(1, 128))
        if i < 2:
            kk = 5000 if i == 1 else 2500
            if i == 0:
                thr = neg_thr
            else:
                vals, _ = lax.top_k(score.reshape(NP), kk)
                thr = vals[kk - 1].reshape(1, 1)
        else:
            vals, _ = lax.top_k(score.reshape(NP), 2500)
            thr = vals[2499].reshape(1, 1)
        xcur = x2

    out = _K5(x2, score, thr,
              params['dec2_W'], params['dec2_b'].reshape(1, 256),
              params['dec1_W'], params['dec1_b'].reshape(1, 512),
              params['dec0_W'], params['dec0_b'].reshape(1, 3750))
    return out.reshape(1, 750, 5)


# speedup vs baseline: 50.7688x; 1.1504x over previous
import functools

import jax
import jax.numpy as jnp
from jax import lax
from jax.experimental import pallas as pl
from jax.experimental.pallas import tpu as pltpu
from jax.experimental.pallas import tpu_sc as plsc

N0 = 10000
NP = 10240
E = 320000
EPAD = 327680
EPS = 1e-5
DUMMY = N0

NC, NS, L = 2, 16, 16
NW = NC * NS
EP = EPAD // NW
BLK = 512
GRID = NP // BLK
SL = NP // NS

_mesh = plsc.VectorSubcoreMesh(core_axis_name="c", subcore_axis_name="s")
_sc_params = pltpu.CompilerParams(needs_layout_passes=False)


_C2 = 2048


@functools.partial(
    pl.kernel, mesh=_mesh,
    out_type=[jax.ShapeDtypeStruct((EPAD,), jnp.int32),
              jax.ShapeDtypeStruct((NC, NP), jnp.float32)],
    scratch_types=[pltpu.VMEM((NP,), jnp.float32),
                   pltpu.VMEM((_C2,), jnp.int32),
                   pltpu.VMEM((_C2,), jnp.int32),
                   pltpu.VMEM((_C2,), jnp.int32),
                   pltpu.VMEM((_C2,), jnp.float32),
                   pltpu.VMEM((SL,), jnp.float32),
                   pltpu.VMEM_SHARED((NP,), jnp.float32)],
    compiler_params=_sc_params,
)
def sc_edge(row_hbm, col_hbm, alive_hbm, rowg_hbm, degp_hbm,
            alive_t, row_v, col_v, rowg_v, ew_v, zero_v, deg_sh):
    c = lax.axis_index("c")
    s = lax.axis_index("s")
    wid = s * NC + c
    base = wid * EP

    pltpu.sync_copy(alive_hbm, alive_t)
    for j in range(SL // L):
        zero_v[pl.ds(j * L, L)] = jnp.zeros((L,), jnp.float32)
    pltpu.sync_copy(zero_v, deg_sh.at[pl.ds(s * SL, SL)])
    plsc.subcore_barrier()

    def chunk(k, _):
        off = base + k * _C2
        pltpu.sync_copy(row_hbm.at[pl.ds(off, _C2)], row_v)
        pltpu.sync_copy(col_hbm.at[pl.ds(off, _C2)], col_v)

        def vec(i, _):
            rv = row_v[pl.ds(i * L, L)]
            cv = col_v[pl.ds(i * L, L)]
            ar = plsc.load_gather(alive_t, [rv])
            ac = plsc.load_gather(alive_t, [cv])
            ok = jnp.logical_and(ar > 0.0, ac > 0.0)
            iota = lax.broadcasted_iota(jnp.int32, (L,), 0)
            dummy = DUMMY + jnp.bitwise_and(iota + i * L, 127)
            rowg_v[pl.ds(i * L, L)] = jnp.where(ok, rv, dummy)
            ew_v[pl.ds(i * L, L)] = jnp.where(ok, 1.0, 0.0)
            return 0

        lax.fori_loop(0, _C2 // L, vec, 0)
        pltpu.sync_copy(rowg_v, rowg_hbm.at[pl.ds(off, _C2)])
        pltpu.sync_copy(ew_v, deg_sh.at[col_v], add=True)
        return 0

    lax.fori_loop(0, EP // _C2, chunk, 0)
    plsc.subcore_barrier()
    pltpu.sync_copy(deg_sh.at[pl.ds(s * SL, SL)],
                    degp_hbm.at[c, pl.ds(s * SL, SL)])


_CHK = 128
_IB = 2048
_NCH = _IB // _CHK
_NB = EP // _IB


@functools.partial(
    pl.kernel, mesh=_mesh,
    out_type=jax.ShapeDtypeStruct((NC, NP, 128), jnp.float32),
    scratch_types=[pltpu.VMEM((_IB,), jnp.int32),
                   pltpu.VMEM((_NCH, _CHK), jnp.int32),
                   pltpu.VMEM((_CHK, 128), jnp.float32),
                   pltpu.VMEM((_CHK, 128), jnp.float32),
                   pltpu.VMEM((32, 128), jnp.float32),
                   pltpu.VMEM_SHARED((NP, 128), jnp.float32),
                   pltpu.SemaphoreType.DMA,
                   pltpu.SemaphoreType.DMA],
    compiler_params=_sc_params,
)
def sc_msg(rowg_hbm, col2d_hbm, g2_hbm, outp_hbm,
           rowg_v, col_v, rows_a, rows_b, zero_v, acc_sh, sem_a, sem_b):
    c = lax.axis_index("c")
    s = lax.axis_index("s")
    wid = s * NC + c
    base = wid * EP

    for i in range(32):
        for j in range(8):
            zero_v[i, pl.ds(j * L, L)] = jnp.zeros((L,), jnp.float32)

    def zrow(jj, _):
        pltpu.sync_copy(zero_v, acc_sh.at[pl.ds(s * SL + jj * 32, 32), :])
        return 0

    lax.fori_loop(0, SL // 32, zrow, 0)
    plsc.subcore_barrier()

    def gissue(j, buf, sem):
        pltpu.async_copy(
            g2_hbm.at[rowg_v.at[pl.ds(j * _CHK, _CHK)]], buf, sem)

    def scat(j, buf):
        pltpu.sync_copy(buf, acc_sh.at[col_v.at[j]], add=True)

    for b in range(_NB):
        off = base + b * _IB
        pltpu.sync_copy(rowg_hbm.at[pl.ds(off, _IB)], rowg_v)
        pltpu.sync_copy(
            col2d_hbm.at[pl.ds(pl.multiple_of(off // _CHK, _NCH), _NCH), :],
            col_v)
        gissue(0, rows_a, sem_a)

        def pair(k2, _):
            j0 = 2 * k2
            pltpu.make_async_copy(g2_hbm.at[rowg_v.at[pl.ds(0, _CHK)]],
                                  rows_a, sem_a).wait()
            gissue(j0 + 1, rows_b, sem_b)
            scat(j0, rows_a)
            pltpu.make_async_copy(g2_hbm.at[rowg_v.at[pl.ds(0, _CHK)]],
                                  rows_b, sem_b).wait()
            gissue(j0 + 2, rows_a, sem_a)
            scat(j0 + 1, rows_b)
            return 0

        lax.fori_loop(0, (_NCH - 2) // 2, pair, 0)
        pltpu.make_async_copy(g2_hbm.at[rowg_v.at[pl.ds(0, _CHK)]],
                              rows_a, sem_a).wait()
        gissue(_NCH - 1, rows_b, sem_b)
        scat(_NCH - 2, rows_a)
        pltpu.make_async_copy(g2_hbm.at[rowg_v.at[pl.ds(0, _CHK)]],
                              rows_b, sem_b).wait()
        scat(_NCH - 1, rows_b)

    plsc.subcore_barrier()
    pltpu.sync_copy(acc_sh.at[pl.ds(s * SL, SL), :],
                    outp_hbm.at[c, pl.ds(s * SL, SL), :])


def _rows_iota():
    return lax.broadcasted_iota(jnp.int32, (BLK, 1), 0)


def _make_k1(first, n_alive):
    def body(x_ref, al_ref, sc_ref, thr_ref, w_ref, b_ref,
             y_ref, st_ref, alout_ref):
        i = pl.program_id(0)
        rows = _rows_iota() + i * BLK
        if first:
            alive = (rows < N0).astype(jnp.float32)
            xs = x_ref[...]
        else:
            sel = jnp.logical_and(al_ref[...] > 0.0,
                                  sc_ref[...] >= thr_ref[0, 0])
            alive = sel.astype(jnp.float32)
            xs = jnp.where(sel, jnp.tanh(sc_ref[...]), 0.0) * x_ref[...]
        y = jnp.dot(xs, w_ref[...], preferred_element_type=jnp.float32)
        y = y + b_ref[...]
        y_ref[...] = y
        alout_ref[...] = alive
        ym = alive * y
        part = jnp.concatenate([jnp.sum(ym, axis=0, keepdims=True),
                                jnp.sum(ym * ym, axis=0, keepdims=True)],
                               axis=0)

        @pl.when(i == 0)
        def _():
            st_ref[...] = part

        @pl.when(i > 0)
        def _():
            st_ref[...] += part

    return pl.pallas_call(
        body,
        grid=(GRID,),
        in_specs=[
            pl.BlockSpec((BLK, 128), lambda i: (i, 0)),
            pl.BlockSpec((BLK, 1), lambda i: (i, 0)),
            pl.BlockSpec((BLK, 1), lambda i: (i, 0)),
            pl.BlockSpec((1, 1), lambda i: (0, 0)),
            pl.BlockSpec((128, 128), lambda i: (0, 0)),
            pl.BlockSpec((1, 128), lambda i: (0, 0)),
        ],
        out_specs=[
            pl.BlockSpec((BLK, 128), lambda i: (i, 0)),
            pl.BlockSpec((2, 128), lambda i: (0, 0)),
            pl.BlockSpec((BLK, 1), lambda i: (i, 0)),
        ],
        out_shape=[
            jax.ShapeDtypeStruct((NP, 128), jnp.float32),
            jax.ShapeDtypeStruct((2, 128), jnp.float32),
            jax.ShapeDtypeStruct((NP, 1), jnp.float32),
        ],
        compiler_params=pltpu.CompilerParams(
            dimension_semantics=("arbitrary",)),
    )


def _dinv_block(degp):
    deg = degp[0] + degp[1] + 1.0
    return 1.0 / jnp.sqrt(deg + 1e-12)


def _make_k2(n_alive):
    def body(y_ref, st_ref, al_ref, dp_ref, g_ref, be_ref, wc_ref, g2_ref):
        m = st_ref[0:1, :] / n_alive
        var = st_ref[1:2, :] / n_alive - m * m
        rstd = 1.0 / jnp.sqrt(var + EPS)
        x1 = jnp.maximum((y_ref[...] - m) * rstd * g_ref[...] + be_ref[...],
                         0.0)
        h = jnp.dot(x1, wc_ref[...], preferred_element_type=jnp.float32)
        g2 = _dinv_block(dp_ref[...]) * h
        g2_ref[...] = jnp.where(al_ref[...] > 0.0, g2, 0.0)

    return pl.pallas_call(
        body,
        grid=(GRID,),
        in_specs=[
            pl.BlockSpec((BLK, 128), lambda i: (i, 0)),
            pl.BlockSpec((2, 128), lambda i: (0, 0)),
            pl.BlockSpec((BLK, 1), lambda i: (i, 0)),
            pl.BlockSpec((2, BLK, 1), lambda i: (0, i, 0)),
            pl.BlockSpec((1, 128), lambda i: (0, 0)),
            pl.BlockSpec((1, 128), lambda i: (0, 0)),
            pl.BlockSpec((128, 128), lambda i: (0, 0)),
        ],
        out_specs=pl.BlockSpec((BLK, 128), lambda i: (i, 0)),
        out_shape=jax.ShapeDtypeStruct((NP, 128), jnp.float32),
        compiler_params=pltpu.CompilerParams(
            dimension_semantics=("arbitrary",)),
    )


def _make_k3():
    def body(op_ref, g2_ref, al_ref, dp_ref, bc_ref, z_ref, st_ref):
        i = pl.program_id(0)
        agg = op_ref[0] + op_ref[1] + g2_ref[...]
        z = _dinv_block(dp_ref[...]) * agg + bc_ref[...]
        z = jnp.where(al_ref[...] > 0.0, z, 0.0)
        z_ref[...] = z
        part = jnp.concatenate([jnp.sum(z, axis=0, keepdims=True),
                                jnp.sum(z * z, axis=0, keepdims=True)],
                               axis=0)

        @pl.when(i == 0)
        def _():
            st_ref[...] = part

        @pl.when(i > 0)
        def _():
            st_ref[...] += part

    return pl.pallas_call(
        body,
        grid=(GRID,),
        in_specs=[
            pl.BlockSpec((2, BLK, 128), lambda i: (0, i, 0)),
            pl.BlockSpec((BLK, 128), lambda i: (i, 0)),
            pl.BlockSpec((BLK, 1), lambda i: (i, 0)),
            pl.BlockSpec((2, BLK, 1), lambda i: (0, i, 0)),
            pl.BlockSpec((1, 128), lambda i: (0, 0)),
        ],
        out_specs=[
            pl.BlockSpec((BLK, 128), lambda i: (i, 0)),
            pl.BlockSpec((2, 128), lambda i: (0, 0)),
        ],
        out_shape=[
            jax.ShapeDtypeStruct((NP, 128), jnp.float32),
            jax.ShapeDtypeStruct((2, 128), jnp.float32),
        ],
        compiler_params=pltpu.CompilerParams(
            dimension_semantics=("arbitrary",)),
    )


def _make_k4(n_alive):
    def body(z_ref, st_ref, al_ref, g_ref, be_ref, p_ref, x2_ref, sc_ref):
        m = st_ref[0:1, :] / n_alive
        var = st_ref[1:2, :] / n_alive - m * m
        rstd = 1.0 / jnp.sqrt(var + EPS)
        x2 = jnp.maximum((z_ref[...] - m) * rstd * g_ref[...] + be_ref[...],
                         0.0)
        alive = al_ref[...] > 0.0
        x2 = jnp.where(alive, x2, 0.0)
        x2_ref[...] = x2
        pn = p_ref[...] / jnp.sqrt(jnp.sum(p_ref[...] * p_ref[...]))
        s = jnp.sum(x2 * pn, axis=1, keepdims=True)
        sc_ref[...] = jnp.where(alive, s, -1e30)

    return pl.pallas_call(
        body,
        grid=(GRID,),
        in_specs=[
            pl.BlockSpec((BLK, 128), lambda i: (i, 0)),
            pl.BlockSpec((2, 128), lambda i: (0, 0)),
            pl.BlockSpec((BLK, 1), lambda i: (i, 0)),
            pl.BlockSpec((1, 128), lambda i: (0, 0)),
            pl.BlockSpec((1, 128), lambda i: (0, 0)),
            pl.BlockSpec((1, 128), lambda i: (0, 0)),
        ],
        out_specs=[
            pl.BlockSpec((BLK, 128), lambda i: (i, 0)),
            pl.BlockSpec((BLK, 1), lambda i: (i, 0)),
        ],
        out_shape=[
            jax.ShapeDtypeStruct((NP, 128), jnp.float32),
            jax.ShapeDtypeStruct((NP, 1), jnp.float32),
        ],
        compiler_params=pltpu.CompilerParams(
            dimension_semantics=("arbitrary",)),
    )


def _make_k5(n_final):
    def body(x2_ref, sc_ref, thr_ref, w2_ref, b2_ref, w1_ref, b1_ref,
             w0_ref, b0_ref, out_ref):
        sel = sc_ref[...] >= thr_ref[0, 0]
        w = jnp.where(sel, jnp.tanh(sc_ref[...]), 0.0)
        xm = jnp.sum(w * x2_ref[...], axis=0, keepdims=True) / n_final
        h = jnp.maximum(jnp.dot(xm, w2_ref[...],
                                preferred_element_type=jnp.float32)
                        + b2_ref[...], 0.0)
        h = jnp.maximum(jnp.dot(h, w1_ref[...],
                                preferred_element_type=jnp.float32)
                        + b1_ref[...], 0.0)
        out_ref[...] = jnp.dot(h, w0_ref[...],
                               preferred_element_type=jnp.float32) + b0_ref[...]

    return pl.pallas_call(
        body,
        out_shape=jax.ShapeDtypeStruct((1, 3750), jnp.float32),
        compiler_params=pltpu.CompilerParams(
            vmem_limit_bytes=100 * 1024 * 1024),
    )


_K1F = _make_k1(True, 10000.0)
_K1A = _make_k1(False, 10000.0)
_K1B = _make_k1(False, 5000.0)
_K2 = {10000.0: _make_k2(10000.0), 5000.0: _make_k2(5000.0)}
_K3 = _make_k3()
_K4 = {10000.0: _make_k4(10000.0), 5000.0: _make_k4(5000.0)}
_K5 = _make_k5(2500.0)

_ZPAD = (0, NP - N0)


def kernel(x, edge_index, params):
    row = jnp.pad(edge_index[0].astype(jnp.int32), (0, EPAD - E),
                  constant_values=N0)
    col = jnp.pad(edge_index[1].astype(jnp.int32), (0, EPAD - E))
    col2d = col.reshape(EPAD // _CHK, _CHK)
    xp = jnp.pad(x, (_ZPAD, (0, 0)))
    alive0 = jnp.pad(jnp.ones((N0, 1), jnp.float32), (_ZPAD, (0, 0)))

    rowg01, degp01 = sc_edge(row, col, alive0.reshape(NP))

    zero_s = jnp.zeros((NP, 1), jnp.float32)
    neg_thr = jnp.full((1, 1), -1e29, jnp.float32)

    xcur = xp
    alive = alive0
    score = zero_s
    thr = neg_thr
    rowg, degp = rowg01, degp01
    n_alive = [10000.0, 10000.0, 5000.0]

    for i in range(3):
        na = n_alive[i]
        k1 = _K1F if i == 0 else (_K1A if i == 1 else _K1B)
        y, st1, alive = k1(xcur, alive, score, thr,
                           params['enc%d_W1' % i],
                           params['enc%d_b1' % i].reshape(1, 128))
        if i == 2:
            rowg, degp = sc_edge(row, col, alive.reshape(NP))
        degp3 = degp.reshape(2, NP, 1)
        g2 = _K2[na](y, st1, alive, degp3,
                     params['enc%d_g1' % i].reshape(1, 128),
                     params['enc%d_be1' % i].reshape(1, 128),
                     params['enc%d_Wc' % i])
        outp = sc_msg(rowg, col2d, g2)
        z, st2 = _K3(outp, g2, alive, degp3,
                     params['enc%d_bc' % i].reshape(1, 128))
        x2, score = _K4[na](z, st2, alive,
                            params['enc%d_g2' % i].reshape(1, 128),
                            params['enc%d_be2' % i].reshape(1, 128),
                            params['enc%d_p' % i].reshape(1, 128))
        if i < 2:
            kk = 5000 if i == 1 else 2500
            if i == 0:
                thr = neg_thr
            else:
                vals, _ = lax.top_k(score.reshape(NP), kk)
                thr = vals[kk - 1].reshape(1, 1)
        else:
            vals, _ = lax.top_k(score.reshape(NP), 2500)
            thr = vals[2499].reshape(1, 1)
        xcur = x2

    out = _K5(x2, score, thr,
              params['dec2_W'], params['dec2_b'].reshape(1, 256),
              params['dec1_W'], params['dec1_b'].reshape(1, 512),
              params['dec0_W'], params['dec0_b'].reshape(1, 3750))
    return out.reshape(1, 750, 5)
